# baseline (speedup 1.0000x reference)
import functools
import jax
import jax.numpy as jnp
from jax import lax
from jax.experimental import pallas as pl
from jax.experimental.pallas import tpu as pltpu
from jax.experimental.pallas import tpu_sc as plsc

_NO = 8
_L = 16
_SUB = _L * _NO
_BIG = 8
_HALF = 4


def _sc_align(x, *, n_workers, unroll):
    n_rows, n_cols = x.shape
    rows_per_worker = n_rows // n_workers
    n_big = rows_per_worker // _BIG
    n_pairs = n_big // 2
    sub_per_row = n_cols // _SUB
    n_sub = _HALF * sub_per_row

    mesh = plsc.VectorSubcoreMesh(core_axis_name="c", subcore_axis_name="s")

    @functools.partial(
        pl.kernel,
        mesh=mesh,
        out_type=jax.ShapeDtypeStruct((n_rows, n_cols), jnp.float32),
        scratch_types=[
            pltpu.VMEM((_BIG, n_cols), jnp.float32),
            pltpu.VMEM((_BIG, n_cols), jnp.float32),
            pltpu.VMEM((_HALF, n_cols), jnp.float32),
            pltpu.VMEM((_HALF, n_cols), jnp.float32),
            pltpu.SemaphoreType.DMA,
            pltpu.SemaphoreType.DMA,
            pltpu.SemaphoreType.DMA,
            pltpu.SemaphoreType.DMA,
        ],
        compiler_params=pltpu.CompilerParams(needs_layout_passes=False),
    )
    def k(x_hbm, out_hbm, inA, inB, out0, out1, isemA, isemB, osem0, osem1):
        nc = lax.axis_size("c")
        wid = lax.axis_index("s") * nc + lax.axis_index("c")
        base = wid * rows_per_worker

        iota = lax.iota(jnp.int32, _L)
        col0 = [iota * _NO + o for o in range(_NO)]

        def start_in(t, buf, sem):
            pltpu.async_copy(
                x_hbm.at[pl.ds(base + t * _BIG, _BIG), :], buf, sem)

        def wait_in(t, buf, sem):
            pltpu.make_async_copy(
                x_hbm.at[pl.ds(base + t * _BIG, _BIG), :], buf, sem).wait()

        def start_out(t, s, buf, sem):
            pltpu.async_copy(
                buf,
                out_hbm.at[pl.ds(base + t * _BIG + s * _HALF, _HALF), :],
                sem)

        def wait_out(buf, sem):
            pltpu.make_async_copy(
                buf, out_hbm.at[pl.ds(base, _HALF), :], sem).wait()

        def compute(in_buf, row_off, out_buf):
            @plsc.parallel_loop(0, n_sub, unroll=unroll)
            def _(c):
                r = c // sub_per_row
                coff = (c % sub_per_row) * _SUB
                src = in_buf.at[row_off + r, pl.ds(coff, _SUB)]
                dst = out_buf.at[r, pl.ds(coff, _SUB)]
                v = [plsc.load_gather(src, [col0[o]])
                     for o in range(_NO)]
                m01 = jnp.maximum(v[0], v[1])
                m23 = jnp.maximum(v[2], v[3])
                m45 = jnp.maximum(v[4], v[5])
                m67 = jnp.maximum(v[6], v[7])
                m03 = jnp.maximum(m01, m23)
                m47 = jnp.maximum(m45, m67)
                m = jnp.maximum(m03, m47)
                take = m03 < m
                y = [jnp.where(take, v[(o + 4) % _NO], v[o])
                     for o in range(_NO)]
                take = jnp.maximum(y[0], y[1]) < m
                y = [jnp.where(take, y[(o + 2) % _NO], y[o])
                     for o in range(_NO)]
                take = y[0] < m
                y = [jnp.where(take, y[(o + 1) % _NO], y[o])
                     for o in range(_NO)]
                for o in range(_NO):
                    plsc.store_scatter(dst, [col0[o]], y[o])

        def pair_body(p, carry):
            tA = 2 * p
            tB = tA + 1
            start_in(tB, inB, isemB)
            wait_in(tA, inA, isemA)

            @pl.when(p > 0)
            def _():
                wait_out(out0, osem0)

            compute(inA, 0, out0)
            start_out(tA, 0, out0, osem0)

            @pl.when(p > 0)
            def _():
                wait_out(out1, osem1)

            compute(inA, _HALF, out1)
            start_out(tA, 1, out1, osem1)

            @pl.when(p + 1 < n_pairs)
            def _():
                start_in(tA + 2, inA, isemA)

            wait_in(tB, inB, isemB)
            wait_out(out0, osem0)
            compute(inB, 0, out0)
            start_out(tB, 0, out0, osem0)
            wait_out(out1, osem1)
            compute(inB, _HALF, out1)
            start_out(tB, 1, out1, osem1)
            return carry

        start_in(0, inA, isemA)
        lax.fori_loop(0, n_pairs, pair_body, None)
        wait_out(out0, osem0)
        wait_out(out1, osem1)

    return k(x)


def kernel(input):
    return _sc_align(input, n_workers=32, unroll=1)
